# Initial kernel scaffold; baseline (speedup 1.0000x reference)
import jax
import jax.numpy as jnp
from jax.experimental import pallas as pl


def kernel(node_features, edge_index, edge_features, current_state, params):
    raise NotImplementedError("write your pallas kernel here")


# speedup vs baseline: 3.8677x; 3.8677x over previous
import jax
import jax.numpy as jnp
from jax import lax
from jax.experimental import pallas as pl
from jax.experimental.pallas import tpu as pltpu
from jax.experimental.pallas import tpu_sc as plsc

N = 50000
E = 800000
H = 128
HEADS = 8
P = 18

NPAD = 51200
SPAN = 5120
ACC = 5376
TRASH = 5120
NB = 10
NC = 2
NS = 16
NW = NC * NS
EPAD = 802816
EW = EPAD // NW
KE = 32
K3 = 32
ZR = 16

BM = 1024
BME = 2048

_f32 = jnp.float32
_i32 = jnp.int32


def _relu(v):
    return jnp.maximum(v, 0.0)


def _lrelu(v):
    return jnp.where(v >= 0, v, 0.2 * v)


def _row(ref):
    return ref[0:1, :]


def _wide(ee):
    return jnp.concatenate([ee, jnp.zeros((ee.shape[0], 112), _f32)], axis=1)


def _maxacc(ref, part16):
    blk = jnp.concatenate(
        [jnp.broadcast_to(part16[None, :], (8, 16)), jnp.zeros((8, 112), _f32)], axis=1)
    i = pl.program_id(0)

    @pl.when(i == 0)
    def _():
        ref[...] = blk

    @pl.when(i != 0)
    def _():
        ref[...] = jnp.maximum(ref[...], blk)


def _t0_body(nf, wne, bne, w0h, w0e, x0, h0, es0, cp):
    x = _relu(jnp.dot(nf[...], wne[...], preferred_element_type=_f32) + _row(bne))
    x0[...] = x
    h0[...] = jnp.dot(x, w0h[...], preferred_element_type=_f32)
    ee = jnp.dot(x, w0e[...], preferred_element_type=_f32)
    es0[...] = _wide(ee)
    _maxacc(cp, jnp.max(ee, axis=0))


def _t0(nfp, wne, bne, w0h, w0e):
    g = NPAD // BM
    full = lambda shp: pl.BlockSpec(shp, lambda i: (0, 0))
    rows = lambda w: pl.BlockSpec((BM, w), lambda i: (i, 0))
    return pl.pallas_call(
        _t0_body,
        grid=(g,),
        in_specs=[rows(16), full((16, H)), full((8, H)), full((H, H)), full((H, 16))],
        out_specs=[rows(H), rows(H), rows(H), full((8, H))],
        out_shape=[
            jax.ShapeDtypeStruct((NPAD, H), _f32),
            jax.ShapeDtypeStruct((NPAD, H), _f32),
            jax.ShapeDtypeStruct((NPAD, H), _f32),
            jax.ShapeDtypeStruct((8, H), _f32),
        ],
    )(nfp, wne, bne, w0h, w0e)


def _merge_body(has_res, x, nm0, nm1, dd0, dd1, esed, cv, wi, bi, wnh, wne2,
                xn_ref, hn_ref, esn_ref, cpn_ref):
    ee = esed[...]
    exs = jnp.exp(_lrelu(ee[:, :8] + ee[:, 8:16]) - cv[0:1, :8])
    h = jnp.dot(x[...], wi[...], preferred_element_type=_f32)
    bm = h.shape[0]
    num = (nm0[...] + nm1[...]).reshape(bm, 8, 16) \
        + h.reshape(bm, 8, 16) * exs[:, :, None]
    den = dd0[:, :8] + dd1[:, :8] + exs
    g = (num / (den[:, :, None] + 1e-16)).reshape(bm, H)
    xn = _relu(g + _row(bi))
    if has_res:
        xn = xn + x[...]
    xn_ref[...] = xn
    hn_ref[...] = jnp.dot(xn, wnh[...], preferred_element_type=_f32)
    ee2 = jnp.dot(xn, wne2[...], preferred_element_type=_f32)
    esn_ref[...] = _wide(ee2)
    _maxacc(cpn_ref, jnp.max(ee2, axis=0))


def _merge(i, hn_width, x, nm0, nm1, dd0, dd1, esed, carr, wi, bi, wnh, wne2):
    import functools
    g = NPAD // BM
    full = lambda shp: pl.BlockSpec(shp, lambda j: (0, 0))
    rows = lambda w: pl.BlockSpec((BM, w), lambda j: (j, 0))
    return pl.pallas_call(
        functools.partial(_merge_body, i > 0),
        grid=(g,),
        in_specs=[rows(H), rows(H), rows(H), rows(H), rows(H), rows(H),
                  full((8, 16)), full((H, H)), full((8, H)),
                  full((H, hn_width)), full((H, 16))],
        out_specs=[rows(H), rows(hn_width), rows(H), full((8, H))],
        out_shape=[
            jax.ShapeDtypeStruct((NPAD, H), _f32),
            jax.ShapeDtypeStruct((NPAD, hn_width), _f32),
            jax.ShapeDtypeStruct((NPAD, H), _f32),
            jax.ShapeDtypeStruct((8, H), _f32),
        ],
    )(x, nm0, nm1, dd0, dd1, esed, carr, wi, bi, wnh, wne2)


def _mden_body(dd0, dd1, esed, cv, out):
    ee = esed[...]
    exs = jnp.exp(_lrelu(ee[:, :8] + ee[:, 8:16]) - cv[0:1, :8])
    den = dd0[:, :8] + dd1[:, :8] + exs + 1e-16
    out[...] = jnp.concatenate(
        [den, ee[:, 8:16], jnp.zeros((den.shape[0], 112), _f32)], axis=1)


def _merge_den(dd0, dd1, esed, carr):
    g = NPAD // BM
    full = lambda shp: pl.BlockSpec(shp, lambda j: (0, 0))
    rows = lambda w: pl.BlockSpec((BM, w), lambda j: (j, 0))
    return pl.pallas_call(
        _mden_body,
        grid=(g,),
        in_specs=[rows(H), rows(H), rows(H), full((8, 16))],
        out_specs=rows(H),
        out_shape=jax.ShapeDtypeStruct((NPAD, H), _f32),
    )(dd0, dd1, esed, carr)


def _m3_body(nm0, nm1, x3, esed, dened, cv, w3, b3, w1a, w1b,
             x4_ref, xs_ref, xd_ref):
    ee = esed[...]
    exs = jnp.exp(_lrelu(ee[:, :8] + ee[:, 8:16]) - cv[0:1, :8])
    alph = exs / dened[:, :8]
    bm = x3.shape[0]
    h3 = jnp.dot(x3[...], w3[...], preferred_element_type=_f32).reshape(bm, 8, H)
    nself = jnp.sum(h3 * alph[:, :, None], axis=1)
    out = (nm0[...] + nm1[...] + nself) / 8.0 + _row(b3)
    x4 = _relu(out) + x3[...]
    x4_ref[...] = x4
    xs_ref[...] = jnp.dot(x4, w1b[...], preferred_element_type=_f32)
    xd_ref[...] = jnp.dot(x4, w1a[...], preferred_element_type=_f32)


def _merge3(nm0, nm1, x3, esed, dened, carr, w3, b3, w1a, w1b):
    g = NPAD // BM
    full = lambda shp: pl.BlockSpec(shp, lambda j: (0, 0))
    rows = lambda w: pl.BlockSpec((BM, w), lambda j: (j, 0))
    return pl.pallas_call(
        _m3_body,
        grid=(g,),
        in_specs=[rows(H), rows(H), rows(H), rows(H), rows(H), full((8, 16)),
                  full((H, 8 * H)), full((8, H)), full((H, H)), full((H, H))],
        out_specs=[rows(H), rows(H), rows(H)],
        out_shape=[jax.ShapeDtypeStruct((NPAD, H), _f32)] * 3,
    )(nm0, nm1, x3, esed, dened, carr, w3, b3, w1a, w1b)


def _ec_body(ef, wee, bee, w1c, mb1, out):
    a = _relu(jnp.dot(ef[...], wee[...], preferred_element_type=_f32) + _row(bee))
    out[...] = jnp.dot(a, w1c[...], preferred_element_type=_f32) + _row(mb1)


def _ec(efp, wee, bee, w1c, mb1):
    g = EPAD // BME
    full = lambda shp: pl.BlockSpec(shp, lambda j: (0, 0))
    rows = lambda w: pl.BlockSpec((BME, w), lambda j: (j, 0))
    return pl.pallas_call(
        _ec_body,
        grid=(g,),
        in_specs=[rows(8), full((8, 32)), full((8, 32)), full((32, H)), full((8, H))],
        out_specs=rows(H),
        out_shape=jax.ShapeDtypeStruct((EPAD, H), _f32),
    )(efp, wee, bee, w1c, mb1)


def _f1_body(x4, nm0, nm1, dd0, dd1, w2, b2, u1a, u1b, ub1, u2, ub2,
             lng, lnb, cw1, cb1, cw2, cb2, xf_ref, cap_ref, xs_ref):
    deg = dd0[:, 0:1] + dd1[:, 0:1]
    rsum = nm0[...] + nm1[...]
    aggr = (jnp.dot(rsum, w2[...], preferred_element_type=_f32)
            + deg * _row(b2)) / jnp.maximum(deg, 1.0)
    x = x4[...]
    u = jnp.dot(
        _relu(jnp.dot(x, u1a[...], preferred_element_type=_f32)
              + jnp.dot(aggr, u1b[...], preferred_element_type=_f32) + _row(ub1)),
        u2[...], preferred_element_type=_f32) + _row(ub2)
    v = u + x
    mu = jnp.mean(v, axis=-1, keepdims=True)
    var = jnp.mean((v - mu) ** 2, axis=-1, keepdims=True)
    xf = (v - mu) / jnp.sqrt(var + 1e-5) * _row(lng) + _row(lnb)
    xf_ref[...] = xf
    t = jnp.dot(_relu(jnp.dot(xf, cw1[...], preferred_element_type=_f32) + _row(cb1)),
                cw2[...], preferred_element_type=_f32) + _row(cb2)
    cap_ref[...] = 1.0 / (1.0 + jnp.exp(-t))
    bm = x.shape[0]
    rid = lax.broadcasted_iota(_i32, (bm, 1), 0) + pl.program_id(0) * bm
    part = jnp.sum(jnp.where(rid < N, xf, 0.0), axis=0)
    i = pl.program_id(0)

    @pl.when(i == 0)
    def _():
        xs_ref[...] = jnp.broadcast_to(part[None, :], (8, H))

    @pl.when(i != 0)
    def _():
        xs_ref[...] = xs_ref[...] + jnp.broadcast_to(part[None, :], (8, H))


def _f1(x4, nm0, nm1, dd0, dd1, w2, b2, u1a, u1b, ub1, u2, ub2, lng, lnb,
        cw1, cb1, cw2, cb2):
    g = NPAD // BM
    full = lambda shp: pl.BlockSpec(shp, lambda j: (0, 0))
    rows = lambda w: pl.BlockSpec((BM, w), lambda j: (j, 0))
    return pl.pallas_call(
        _f1_body,
        grid=(g,),
        in_specs=[rows(H), rows(H), rows(H), rows(H), rows(H),
                  full((H, H)), full((8, H)), full((H, H)), full((H, H)),
                  full((8, H)), full((H, H)), full((8, H)), full((8, H)),
                  full((8, H)), full((H, 64)), full((8, 64)), full((64, H)),
                  full((8, H))],
        out_specs=[rows(H), rows(H), full((8, H))],
        out_shape=[
            jax.ShapeDtypeStruct((NPAD, H), _f32),
            jax.ShapeDtypeStruct((NPAD, H), _f32),
            jax.ShapeDtypeStruct((8, H), _f32),
        ],
    )(x4, nm0, nm1, dd0, dd1, w2, b2, u1a, u1b, ub1, u2, ub2, lng, lnb,
      cw1, cb1, cw2, cb2)


def _f2_body(xsp, cs, gpw, gpb, a1a, a1b, ab1, aw2, ab2, act_ref, gr_ref):
    gsum = xsp[0:1, :] * (1.0 / N)
    g0 = _relu(jnp.dot(gsum, gpw[...], preferred_element_type=_f32) + _row(gpb))
    gr_ref[...] = jnp.broadcast_to(g0, (8, H))
    ga = jnp.dot(g0, a1a[...], preferred_element_type=_f32)
    comb = _relu(jnp.dot(cs[...], a1b[...], preferred_element_type=_f32)
                 + ga + _row(ab1))
    act_ref[...] = jnp.dot(comb, aw2[...], preferred_element_type=_f32) + _row(ab2)


def _f2(xsp, cs, gpw, gpb, a1a, a1b, ab1, aw2, ab2):
    full = lambda shp: pl.BlockSpec(shp, lambda: (0, 0))
    return pl.pallas_call(
        _f2_body,
        in_specs=[full((8, H)), full((1024, H)), full((H, H)), full((8, H)),
                  full((H, H)), full((H, H)), full((8, H)), full((H, H)),
                  full((8, H))],
        out_specs=[full((1024, H)), full((8, H))],
        out_shape=[
            jax.ShapeDtypeStruct((1024, H), _f32),
            jax.ShapeDtypeStruct((8, H), _f32),
        ],
    )(xsp, cs, gpw, gpb, a1a, a1b, ab1, aw2, ab2)


_GDN = lax.GatherDimensionNumbers(
    offset_dims=(), collapsed_slice_dims=(0,), start_index_map=(0,))


def _lane_pick(v, idx16):
    return lax.gather(v, idx16[:, None], _GDN, (1,),
                      mode=lax.GatherScatterMode.PROMISE_IN_BOUNDS)


def _lane_bcast(v, k):
    return _lane_pick(v, jnp.full((16,), k, _i32))


def _rot8(v):
    idx = jnp.bitwise_and(lax.iota(_i32, 16) + 8, 15)
    return _lane_pick(v, idx)


def _worker_ids():
    c = lax.axis_index("c")
    s = lax.axis_index("s")
    return c, s, c * NS + s


def _fill_zeros(bufs):
    zv = jnp.zeros((16,), _f32)
    for buf in bufs:
        rows, width = buf.shape

        def zf(r, _):
            for col in range(width // 16):
                buf[r, pl.ds(col * 16, 16)] = zv
            return 0

        lax.fori_loop(0, rows, zf, 0)


def _zero_acc(s, accs):
    for ch in range(21):
        base = s * 336 + ch * ZR
        for acc, zb in accs:
            pltpu.sync_copy(zb, acc.at[pl.ds(base, ZR)])


def _sidx_fill(dbuf, sidx, b, k):
    def sx(t, _):
        dv = dbuf[pl.ds(t * 16, 16)]
        lv = dv - b * SPAN
        ok = (lv >= 0) & (lv < SPAN)
        sidx[pl.ds(t * 16, 16)] = jnp.where(ok, lv, TRASH)
        return 0

    lax.fori_loop(0, k // 16, sx, 0)


def _writeout(c, s, b, pairs):
    for acc, out in pairs:
        pltpu.sync_copy(acc.at[pl.ds(s * 320, 320)],
                        out.at[c, pl.ds(b * SPAN + s * 320, 320)])


_SC_MESH = dict(core_axis_name="c", subcore_axis_name="s",
                num_cores=NC, num_subcores=NS)


def _sc_gat(htab, esedtab, cvec, srcs, dsts):
    def body(htab, esedtab, cvec, srcs, dsts, nump, denp,
             accn, accd, sbuf, dbuf, sidx, hrows, e1r, e2r, onum, oden,
             cvv, zb, sem1, sem2, sem3):
        c, s, w = _worker_ids()
        _fill_zeros([zb, oden])
        pltpu.sync_copy(cvec, cvv)
        cc = cvv[...]
        for b in range(NB):
            _zero_acc(s, [(accn, zb), (accd, zb)])
            plsc.subcore_barrier()

            def batch(j, _):
                base = w * EW + j * KE
                pltpu.sync_copy(srcs.at[pl.ds(base, KE)], sbuf)
                pltpu.sync_copy(dsts.at[pl.ds(base, KE)], dbuf)
                _sidx_fill(dbuf, sidx, b, KE)
                cp1 = pltpu.async_copy(htab.at[sbuf], hrows, sem1)
                cp2 = pltpu.async_copy(esedtab.at[sbuf], e1r, sem2)
                cp3 = pltpu.async_copy(esedtab.at[dbuf], e2r, sem3)
                cp1.wait(); cp2.wait(); cp3.wait()

                def pe(i, _):
                    ev = e1r[i, pl.ds(0, 16)] + _rot8(e2r[i, pl.ds(0, 16)])
                    exv = jnp.exp(_lrelu(ev) - cc)
                    for r in range(8):
                        al = _lane_bcast(exv, r)
                        onum[i, pl.ds(r * 16, 16)] = hrows[i, pl.ds(r * 16, 16)] * al
                    oden[i, pl.ds(0, 16)] = exv
                    return 0

                lax.fori_loop(0, KE, pe, 0)
                pltpu.sync_copy(onum, accn.at[sidx], add=True)
                pltpu.sync_copy(oden, accd.at[sidx], add=True)
                return 0

            lax.fori_loop(0, EW // KE, batch, 0)
            plsc.subcore_barrier()
            _writeout(c, s, b, [(accn, nump), (accd, denp)])
            plsc.subcore_barrier()

    return pl.kernel(
        body,
        out_type=[jax.ShapeDtypeStruct((NC, NPAD, H), _f32),
                  jax.ShapeDtypeStruct((NC, NPAD, H), _f32)],
        mesh=plsc.VectorSubcoreMesh(**_SC_MESH),
        scratch_types=[
            pltpu.VMEM_SHARED((ACC, H), _f32),
            pltpu.VMEM_SHARED((ACC, H), _f32),
            pltpu.VMEM((KE,), _i32), pltpu.VMEM((KE,), _i32), pltpu.VMEM((KE,), _i32),
            pltpu.VMEM((KE, H), _f32),
            pltpu.VMEM((KE, H), _f32), pltpu.VMEM((KE, H), _f32),
            pltpu.VMEM((KE, H), _f32), pltpu.VMEM((KE, H), _f32),
            pltpu.VMEM((16,), _f32),
            pltpu.VMEM((ZR, H), _f32),
            pltpu.SemaphoreType.DMA, pltpu.SemaphoreType.DMA, pltpu.SemaphoreType.DMA,
        ],
    )(htab, esedtab, cvec, srcs, dsts)


def _sc_den(esedtab, cvec, srcs, dsts):
    def body(esedtab, cvec, srcs, dsts, denp,
             accd, sbuf, dbuf, sidx, e1r, e2r, oden, cvv, zb, sem2, sem3):
        c, s, w = _worker_ids()
        _fill_zeros([zb, oden])
        pltpu.sync_copy(cvec, cvv)
        cc = cvv[...]
        for b in range(NB):
            _zero_acc(s, [(accd, zb)])
            plsc.subcore_barrier()

            def batch(j, _):
                base = w * EW + j * KE
                pltpu.sync_copy(srcs.at[pl.ds(base, KE)], sbuf)
                pltpu.sync_copy(dsts.at[pl.ds(base, KE)], dbuf)
                _sidx_fill(dbuf, sidx, b, KE)
                cp2 = pltpu.async_copy(esedtab.at[sbuf], e1r, sem2)
                cp3 = pltpu.async_copy(esedtab.at[dbuf], e2r, sem3)
                cp2.wait(); cp3.wait()

                def pe(i, _):
                    ev = e1r[i, pl.ds(0, 16)] + _rot8(e2r[i, pl.ds(0, 16)])
                    oden[i, pl.ds(0, 16)] = jnp.exp(_lrelu(ev) - cc)
                    return 0

                lax.fori_loop(0, KE, pe, 0)
                pltpu.sync_copy(oden, accd.at[sidx], add=True)
                return 0

            lax.fori_loop(0, EW // KE, batch, 0)
            plsc.subcore_barrier()
            _writeout(c, s, b, [(accd, denp)])
            plsc.subcore_barrier()

    return pl.kernel(
        body,
        out_type=jax.ShapeDtypeStruct((NC, NPAD, H), _f32),
        mesh=plsc.VectorSubcoreMesh(**_SC_MESH),
        scratch_types=[
            pltpu.VMEM_SHARED((ACC, H), _f32),
            pltpu.VMEM((KE,), _i32), pltpu.VMEM((KE,), _i32), pltpu.VMEM((KE,), _i32),
            pltpu.VMEM((KE, H), _f32), pltpu.VMEM((KE, H), _f32),
            pltpu.VMEM((KE, H), _f32),
            pltpu.VMEM((16,), _f32),
            pltpu.VMEM((ZR, H), _f32),
            pltpu.SemaphoreType.DMA, pltpu.SemaphoreType.DMA,
        ],
    )(esedtab, cvec, srcs, dsts)


def _sc_g3(h3tab, esedtab, denedtab, cvec, srcs, dsts):
    def body(h3tab, esedtab, denedtab, cvec, srcs, dsts, nump,
             accn, sbuf, dbuf, sidx, hrows, e1r, ddr, onum,
             cvv, zb, sem1, sem2, sem3):
        c, s, w = _worker_ids()
        _fill_zeros([zb])
        pltpu.sync_copy(cvec, cvv)
        cc = cvv[...]
        for b in range(NB):
            _zero_acc(s, [(accn, zb)])
            plsc.subcore_barrier()

            def batch(j, _):
                base = w * EW + j * K3
                pltpu.sync_copy(srcs.at[pl.ds(base, K3)], sbuf)
                pltpu.sync_copy(dsts.at[pl.ds(base, K3)], dbuf)
                _sidx_fill(dbuf, sidx, b, K3)
                cp1 = pltpu.async_copy(h3tab.at[sbuf], hrows, sem1)
                cp2 = pltpu.async_copy(esedtab.at[sbuf], e1r, sem2)
                cp3 = pltpu.async_copy(denedtab.at[dbuf], ddr, sem3)
                cp1.wait(); cp2.wait(); cp3.wait()

                def pe(i, _):
                    dd = ddr[i, pl.ds(0, 16)]
                    ev = e1r[i, pl.ds(0, 16)] + _rot8(dd)
                    alph = jnp.exp(_lrelu(ev) - cc) / dd
                    accs = [None] * 8
                    for k in range(8):
                        ak = _lane_bcast(alph, k)
                        for r in range(8):
                            t = hrows[i, pl.ds(k * H + r * 16, 16)] * ak
                            accs[r] = t if k == 0 else accs[r] + t
                    for r in range(8):
                        onum[i, pl.ds(r * 16, 16)] = accs[r]
                    return 0

                lax.fori_loop(0, K3, pe, 0)
                pltpu.sync_copy(onum, accn.at[sidx], add=True)
                return 0

            lax.fori_loop(0, EW // K3, batch, 0)
            plsc.subcore_barrier()
            _writeout(c, s, b, [(accn, nump)])
            plsc.subcore_barrier()

    return pl.kernel(
        body,
        out_type=jax.ShapeDtypeStruct((NC, NPAD, H), _f32),
        mesh=plsc.VectorSubcoreMesh(**_SC_MESH),
        scratch_types=[
            pltpu.VMEM_SHARED((ACC, H), _f32),
            pltpu.VMEM((K3,), _i32), pltpu.VMEM((K3,), _i32), pltpu.VMEM((K3,), _i32),
            pltpu.VMEM((K3, 8 * H), _f32),
            pltpu.VMEM((K3, H), _f32), pltpu.VMEM((K3, H), _f32),
            pltpu.VMEM((K3, H), _f32),
            pltpu.VMEM((16,), _f32),
            pltpu.VMEM((ZR, H), _f32),
            pltpu.SemaphoreType.DMA, pltpu.SemaphoreType.DMA, pltpu.SemaphoreType.DMA,
        ],
    )(h3tab, esedtab, denedtab, cvec, srcs, dsts)


def _sc_msg(xstab, xdtab, ectab, srcs, dsts):
    def body(xstab, xdtab, ectab, srcs, dsts, nump, denp,
             accn, accd, sbuf, dbuf, sidx, xsr, xdr, ecr, onum, oden, zb,
             sem1, sem2, sem3):
        c, s, w = _worker_ids()
        _fill_zeros([zb, oden])
        onev = jnp.where(lax.iota(_i32, 16) == 0, 1.0, 0.0).astype(_f32)

        def of(i, _):
            oden[i, pl.ds(0, 16)] = onev
            return 0

        lax.fori_loop(0, KE, of, 0)
        for b in range(NB):
            _zero_acc(s, [(accn, zb), (accd, zb)])
            plsc.subcore_barrier()

            def batch(j, _):
                base = w * EW + j * KE
                pltpu.sync_copy(srcs.at[pl.ds(base, KE)], sbuf)
                pltpu.sync_copy(dsts.at[pl.ds(base, KE)], dbuf)
                _sidx_fill(dbuf, sidx, b, KE)
                cp1 = pltpu.async_copy(xstab.at[sbuf], xsr, sem1)
                cp2 = pltpu.async_copy(xdtab.at[dbuf], xdr, sem2)
                cp3 = pltpu.async_copy(ectab.at[pl.ds(base, KE)], ecr, sem3)
                cp1.wait(); cp2.wait(); cp3.wait()

                def pe(i, _):
                    for r in range(8):
                        sl = pl.ds(r * 16, 16)
                        onum[i, sl] = jnp.maximum(
                            xsr[i, sl] + xdr[i, sl] + ecr[i, sl], 0.0)
                    return 0

                lax.fori_loop(0, KE, pe, 0)
                pltpu.sync_copy(onum, accn.at[sidx], add=True)
                pltpu.sync_copy(oden, accd.at[sidx], add=True)
                return 0

            lax.fori_loop(0, EW // KE, batch, 0)
            plsc.subcore_barrier()
            _writeout(c, s, b, [(accn, nump), (accd, denp)])
            plsc.subcore_barrier()

    return pl.kernel(
        body,
        out_type=[jax.ShapeDtypeStruct((NC, NPAD, H), _f32),
                  jax.ShapeDtypeStruct((NC, NPAD, H), _f32)],
        mesh=plsc.VectorSubcoreMesh(**_SC_MESH),
        scratch_types=[
            pltpu.VMEM_SHARED((ACC, H), _f32),
            pltpu.VMEM_SHARED((ACC, H), _f32),
            pltpu.VMEM((KE,), _i32), pltpu.VMEM((KE,), _i32), pltpu.VMEM((KE,), _i32),
            pltpu.VMEM((KE, H), _f32), pltpu.VMEM((KE, H), _f32),
            pltpu.VMEM((KE, H), _f32),
            pltpu.VMEM((KE, H), _f32), pltpu.VMEM((KE, H), _f32),
            pltpu.VMEM((ZR, H), _f32),
            pltpu.SemaphoreType.DMA, pltpu.SemaphoreType.DMA, pltpu.SemaphoreType.DMA,
        ],
    )(xstab, xdtab, ectab, srcs, dsts)


def _b8(v):
    return jnp.broadcast_to(v.reshape(1, -1), (8, v.shape[-1])).astype(_f32)


def _cvecs(cp):
    c = _lrelu(cp[0, :8] + cp[0, 8:16])
    cv = jnp.concatenate([c, jnp.full((8,), 60.0, _f32)])
    return cv, _b8(jnp.concatenate([c, jnp.zeros((8,), _f32)]))


def kernel(node_features, edge_index, edge_features, current_state, params):
    p = params
    srcs = jnp.pad(edge_index[0], (0, EPAD - E))
    dsts = jnp.pad(edge_index[1], (0, EPAD - E), constant_values=N)
    nfp = jnp.pad(node_features.astype(_f32), ((0, NPAD - N), (0, 6)))
    efp = jnp.pad(edge_features.astype(_f32), ((0, EPAD - E), (0, 4)))

    Whs, Wees, bs = [], [], []
    for i in range(4):
        d = 16 if i < 3 else H
        W = p['gat%d_W' % i]
        Wr = W.reshape(H, HEADS, d)
        As = jnp.einsum('hkd,kd->hk', Wr, p['gat%d_as' % i])
        Ad = jnp.einsum('hkd,kd->hk', Wr, p['gat%d_ad' % i])
        Whs.append(W)
        Wees.append(jnp.concatenate([As, Ad], axis=1))
        bs.append(_b8(p['gat%d_b' % i]))

    wne = jnp.pad(p['W_ne'], ((0, 6), (0, 0)))
    x0, h0, esed0, cp0 = _t0(nfp, wne, _b8(p['b_ne']), Whs[0], Wees[0])
    cv, carr = _cvecs(cp0)

    xs = [x0]
    h, esed = h0, esed0
    for i in range(3):
        nm, dd = _sc_gat(h, esed, cv, srcs, dsts)
        hw = 8 * H if i == 2 else H
        xn, h, esed, cp = _merge(i, hw, xs[-1], nm[0], nm[1], dd[0], dd[1],
                                 esed, carr, Whs[i], bs[i], Whs[i + 1], Wees[i + 1])
        cv, carr = _cvecs(cp)
        xs.append(xn)

    dd3 = _sc_den(esed, cv, srcs, dsts)
    dened = _merge_den(dd3[0], dd3[1], esed, carr)
    nm3 = _sc_g3(h, esed, dened, cv, srcs, dsts)
    W1 = p['msg_W1']
    x4, xstab, xdtab = _merge3(nm3[0], nm3[1], xs[-1], esed, dened, carr,
                               Whs[3], bs[3], W1[:H], W1[H:2 * H])

    wee = jnp.pad(p['W_ee'], ((0, 4), (0, 0)))
    ectab = _ec(efp, wee, _b8(p['b_ee']), W1[2 * H:], _b8(p['msg_b1']))
    nmm, ddm = _sc_msg(xstab, xdtab, ectab, srcs, dsts)

    U1 = p['upd_W1']
    cw2 = jnp.pad(p['cp_W2'], ((0, 0), (0, H - P)))
    cb2 = _b8(jnp.pad(p['cp_b2'], (0, H - P)))
    xf, cap, xsp = _f1(x4, nmm[0], nmm[1], ddm[0], ddm[1],
                       p['msg_W2'], _b8(p['msg_b2']), U1[:H], U1[H:],
                       _b8(p['upd_b1']), p['upd_W2'], _b8(p['upd_b2']),
                       _b8(p['ln_g']), _b8(p['ln_b']),
                       p['cp_W1'], _b8(p['cp_b1']), cw2, cb2)

    A1 = p['ah_W1']
    aw2 = jnp.pad(p['ah_W2'], ((0, 0), (0, H - P)))
    ab2 = _b8(jnp.pad(p['ah_b2'], (0, H - P)))
    act, gr8 = _f2(xsp, current_state, p['gp_W'], _b8(p['gp_b']),
                   A1[:H], A1[H:], _b8(p['ah_b1']), aw2, ab2)

    return act[:, :P], cap[:N, :P], xf[:N], gr8[0:1]
